# baseline (speedup 1.0000x reference)
import functools

import jax
import jax.numpy as jnp
from jax import lax
from jax.experimental import pallas as pl
from jax.experimental.pallas import tpu as pltpu
from jax.experimental.pallas import tpu_sc as plsc

_VOCAB = 9487
_L = 16
_ND = 15
_VPT = 40
_CHUNK = _VPT * _L
_PADDED = _ND * _CHUNK
_NTGT = 80

_LOG_COEFFS = (
    7.0376836292e-2, -1.1514610310e-1, 1.1676998740e-1, -1.2420140846e-1,
    1.4249322787e-1, -1.6668057665e-1, 2.0000714765e-1, -2.4999993993e-1,
    3.3333331174e-1,
)
_LN2 = 0.6931471805599453


def _vlog(x):
    bits = lax.bitcast_convert_type(x, jnp.int32)
    e = (bits >> 23) - 126
    m_bits = (bits & 0x007FFFFF) | 0x3F000000
    m = lax.bitcast_convert_type(m_bits, jnp.float32)
    ef = e.astype(jnp.float32)
    adj = m < jnp.float32(0.70710678)
    ef = jnp.where(adj, ef - 1.0, ef)
    t = jnp.where(adj, m + m - 1.0, m - 1.0)
    z = t * t
    p = jnp.float32(_LOG_COEFFS[0])
    for c in _LOG_COEFFS[1:]:
        p = p * t + jnp.float32(c)
    y = p * t * z - 0.5 * z
    return t + y + ef * jnp.float32(_LN2)


def _mil_body(x_hbm, tgt_hbm, out_hbm, xc_v, xf_v, tgt_v, ids_v, acc_v):
    wid = lax.axis_index("s")
    zeros = jnp.zeros((_L,), jnp.float32)
    lane = lax.iota(jnp.int32, _L)

    @pl.when(wid < _ND)
    def _dense():
        pltpu.sync_copy(x_hbm.at[pl.ds(wid * _CHUNK, _CHUNK)], xc_v)

        def body(j, acc):
            x = xc_v[pl.ds(j * _L, _L)]
            return acc + _vlog(1.0 - x + 1e-15)

        neg = lax.fori_loop(0, _VPT, body, zeros)
        acc_v[...] = neg
        pltpu.sync_copy(acc_v, out_hbm.at[pl.ds(wid * _L, _L)])

    @pl.when(wid == _ND)
    def _sparse():
        pltpu.sync_copy(x_hbm, xf_v)
        pltpu.sync_copy(tgt_hbm, tgt_v)
        for j in range(_NTGT // _L):
            occ = j * _L + lane
            idx = tgt_v[pl.ds(j * _L, _L)]
            plsc.store_scatter(ids_v, [idx], occ)
        pos = bcor = cnt = zeros
        for j in range(_NTGT // _L):
            occ = j * _L + lane
            idx = tgt_v[pl.ds(j * _L, _L)]
            rep = plsc.load_gather(ids_v, [idx])
            p = plsc.load_gather(xf_v, [idx])
            m = jnp.where((rep == occ) & (idx > 0), jnp.float32(1.0),
                          jnp.float32(0.0))
            pos = pos + _vlog(p + 1e-30) * m
            bcor = bcor + _vlog(1.0 - p + 1e-15) * m
            cnt = cnt + m
        x0 = xf_v[pl.ds(0, _L)]
        bcor = bcor + jnp.where(lane == 0, _vlog(1.0 - x0 + 1e-15), 0.0)
        acc_v[...] = pos
        pltpu.sync_copy(acc_v, out_hbm.at[pl.ds(_ND * _L, _L)])
        acc_v[...] = bcor
        pltpu.sync_copy(acc_v, out_hbm.at[pl.ds((_ND + 1) * _L, _L)])
        acc_v[...] = cnt
        pltpu.sync_copy(acc_v, out_hbm.at[pl.ds((_ND + 2) * _L, _L)])


_mil_kernel = functools.partial(
    pl.kernel,
    out_type=jax.ShapeDtypeStruct(((_ND + 3) * _L,), jnp.float32),
    mesh=plsc.VectorSubcoreMesh(core_axis_name="c", subcore_axis_name="s",
                                num_cores=1),
    compiler_params=pltpu.CompilerParams(needs_layout_passes=False),
    scratch_types=[
        pltpu.VMEM((_CHUNK,), jnp.float32),
        pltpu.VMEM((_PADDED,), jnp.float32),
        pltpu.VMEM((_NTGT,), jnp.int32),
        pltpu.VMEM((_VOCAB,), jnp.int32),
        pltpu.VMEM((_L,), jnp.float32),
    ],
)(_mil_body)


def kernel(input, target):
    row = input.reshape(-1)
    x = jnp.pad(row, (0, _PADDED - _VOCAB))
    out = _mil_kernel(x, target.reshape(-1))
    neg_all = jnp.sum(out[: _ND * _L])
    pos = jnp.sum(out[_ND * _L : (_ND + 1) * _L])
    bcor = jnp.sum(out[(_ND + 1) * _L : (_ND + 2) * _L])
    cnt = jnp.sum(out[(_ND + 2) * _L :])
    return -pos / cnt - (neg_all - bcor) / (jnp.float32(_VOCAB - 1) - cnt)


# speedup vs baseline: 1.2488x; 1.2488x over previous
import functools

import jax
import jax.numpy as jnp
from jax import lax
from jax.experimental import pallas as pl
from jax.experimental.pallas import tpu as pltpu
from jax.experimental.pallas import tpu_sc as plsc

_VOCAB = 9487
_L = 16
_ND = 14
_VPT = 42
_CHUNK = _VPT * _L
_DENSE = _ND * _CHUNK
_TAILV = 5
_ROWPAD = _DENSE + _TAILV * _L
_NTGT = 80
_NPART = _ND + 3

_LOG_COEFFS = (
    7.0376836292e-2, -1.1514610310e-1, 1.1676998740e-1, -1.2420140846e-1,
    1.4249322787e-1, -1.6668057665e-1, 2.0000714765e-1, -2.4999993993e-1,
    3.3333331174e-1,
)
_LN2 = 0.6931471805599453


def _vlog(x):
    bits = lax.bitcast_convert_type(x, jnp.int32)
    e = (bits >> 23) - 126
    m_bits = (bits & 0x007FFFFF) | 0x3F000000
    m = lax.bitcast_convert_type(m_bits, jnp.float32)
    ef = e.astype(jnp.float32)
    adj = m < jnp.float32(0.70710678)
    ef = jnp.where(adj, ef - 1.0, ef)
    t = jnp.where(adj, m + m - 1.0, m - 1.0)
    z = t * t
    p = jnp.float32(_LOG_COEFFS[0])
    for c in _LOG_COEFFS[1:]:
        p = p * t + jnp.float32(c)
    y = p * t * z - 0.5 * z
    return t + y + ef * jnp.float32(_LN2)


def _neg_term(x):
    return _vlog(1.0 - x + 1e-15)


def _mil_body(x_hbm, tgt_hbm, out_hbm, xc_v, xf_v, tgt_v, ids_v, stage_v,
              gath_v, shared_v):
    wid = lax.axis_index("s")
    zeros = jnp.zeros((_L,), jnp.float32)
    lane = lax.iota(jnp.int32, _L)

    @pl.when(wid < _ND)
    def _dense():
        pltpu.sync_copy(x_hbm.at[pl.ds(wid * _CHUNK, _CHUNK)], xc_v)

        def body(j, acc):
            return acc + _neg_term(xc_v[pl.ds(j * _L, _L)])

        stage_v[...] = lax.fori_loop(0, _VPT, body, zeros)
        pltpu.sync_copy(stage_v, shared_v.at[pl.ds(wid * _L, _L)])

    @pl.when(wid == _ND)
    def _sparse():
        xf_v[pl.ds(_ROWPAD - _L, _L)] = zeros
        pltpu.sync_copy(x_hbm, xf_v.at[pl.ds(0, _VOCAB)])
        pltpu.sync_copy(tgt_hbm, tgt_v)
        for j in range(_NTGT // _L):
            idx = tgt_v[pl.ds(j * _L, _L)]
            plsc.store_scatter(ids_v, [idx], j * _L + lane)
        pos = bcor = cnt = zeros
        for j in range(_NTGT // _L):
            occ = j * _L + lane
            idx = tgt_v[pl.ds(j * _L, _L)]
            rep = plsc.load_gather(ids_v, [idx])
            p = plsc.load_gather(xf_v, [idx])
            m = jnp.where((rep == occ) & (idx > 0), jnp.float32(1.0),
                          jnp.float32(0.0))
            pos = pos + _vlog(p + 1e-30) * m
            bcor = bcor + _neg_term(p) * m
            cnt = cnt + m
        bcor = bcor + jnp.where(lane == 0, _neg_term(xf_v[pl.ds(0, _L)]), 0.0)
        tail = zeros
        for j in range(_TAILV):
            tail = tail + _neg_term(xf_v[pl.ds(_DENSE + j * _L, _L)])
        stage_v[...] = pos
        pltpu.sync_copy(stage_v, shared_v.at[pl.ds(_ND * _L, _L)])
        stage_v[...] = bcor
        pltpu.sync_copy(stage_v, shared_v.at[pl.ds((_ND + 1) * _L, _L)])
        stage_v[...] = cnt
        pltpu.sync_copy(stage_v, shared_v.at[pl.ds((_ND + 2) * _L, _L)])
        stage_v[...] = tail
        pltpu.sync_copy(stage_v, shared_v.at[pl.ds((_ND + 3) * _L, _L)])

    plsc.subcore_barrier()

    @pl.when(wid == 0)
    def _finalize():
        pltpu.sync_copy(shared_v, gath_v)
        neg = zeros
        for t in range(_ND):
            neg = neg + gath_v[pl.ds(t * _L, _L)]
        neg = neg + gath_v[pl.ds((_ND + 3) * _L, _L)]
        pos_s = jnp.full((_L,), jnp.sum(gath_v[pl.ds(_ND * _L, _L)]))
        bcor_s = jnp.full((_L,), jnp.sum(gath_v[pl.ds((_ND + 1) * _L, _L)]))
        cnt_s = jnp.full((_L,), jnp.sum(gath_v[pl.ds((_ND + 2) * _L, _L)]))
        neg_s = jnp.full((_L,), jnp.sum(neg))
        loss = (-pos_s / cnt_s
                - (neg_s - bcor_s) / (jnp.float32(_VOCAB - 1) - cnt_s))
        stage_v[...] = loss
        pltpu.sync_copy(stage_v.at[pl.ds(0, 1)], out_hbm)


_mil_kernel = functools.partial(
    pl.kernel,
    out_type=jax.ShapeDtypeStruct((1,), jnp.float32),
    mesh=plsc.VectorSubcoreMesh(core_axis_name="c", subcore_axis_name="s",
                                num_cores=1),
    compiler_params=pltpu.CompilerParams(needs_layout_passes=False),
    scratch_types=[
        pltpu.VMEM((_CHUNK,), jnp.float32),
        pltpu.VMEM((_ROWPAD,), jnp.float32),
        pltpu.VMEM((_NTGT,), jnp.int32),
        pltpu.VMEM((_VOCAB,), jnp.int32),
        pltpu.VMEM((_L,), jnp.float32),
        pltpu.VMEM(((_NPART + 1) * _L,), jnp.float32),
        pltpu.VMEM_SHARED(((_NPART + 1) * _L,), jnp.float32),
    ],
)(_mil_body)


def kernel(input, target):
    out = _mil_kernel(input.reshape(-1), target.reshape(-1))
    return out.reshape(())


# speedup vs baseline: 1.2856x; 1.0295x over previous
import functools

import jax
import jax.numpy as jnp
from jax import lax
from jax.experimental import pallas as pl
from jax.experimental.pallas import tpu as pltpu
from jax.experimental.pallas import tpu_sc as plsc

_VOCAB = 9487
_L = 16
_ND = 14
_VPT = 42
_CHUNK = _VPT * _L
_DENSE = _ND * _CHUNK
_TAILV = 5
_ROWPAD = _DENSE + _TAILV * _L
_NTGT = 80
_NPART = _ND + 3

_LOG_COEFFS = (
    7.0376836292e-2, -1.1514610310e-1, 1.1676998740e-1, -1.2420140846e-1,
    1.4249322787e-1, -1.6668057665e-1, 2.0000714765e-1, -2.4999993993e-1,
    3.3333331174e-1,
)
_LN2 = 0.6931471805599453


def _vlog(x):
    bits = lax.bitcast_convert_type(x, jnp.int32)
    e = (bits >> 23) - 126
    m_bits = (bits & 0x007FFFFF) | 0x3F000000
    m = lax.bitcast_convert_type(m_bits, jnp.float32)
    ef = e.astype(jnp.float32)
    adj = m < jnp.float32(0.70710678)
    ef = jnp.where(adj, ef - 1.0, ef)
    t = jnp.where(adj, m + m - 1.0, m - 1.0)
    z = t * t
    p = jnp.float32(_LOG_COEFFS[0])
    for c in _LOG_COEFFS[1:]:
        p = p * t + jnp.float32(c)
    y = p * t * z - 0.5 * z
    return t + y + ef * jnp.float32(_LN2)


def _neg_term(x):
    return _vlog(1.0 - x + 1e-15)


def _mil_body(x_hbm, tgt_hbm, out_hbm, xc_v, xf_v, tgt_v, ids_v, stage_v,
              pack_v, gath_v, shared_v, row_sem):
    wid = lax.axis_index("s")
    zeros = jnp.zeros((_L,), jnp.float32)
    lane = lax.iota(jnp.int32, _L)

    @pl.when(wid < _ND)
    def _dense():
        pltpu.sync_copy(x_hbm.at[pl.ds(wid * _CHUNK, _CHUNK)], xc_v)

        def body(j, acc):
            return acc + _neg_term(xc_v[pl.ds(j * _L, _L)])

        stage_v[...] = lax.fori_loop(0, _VPT, body, zeros)
        pltpu.sync_copy(stage_v, shared_v.at[pl.ds(wid * _L, _L)])

    @pl.when(wid == _ND)
    def _sparse():
        xf_v[pl.ds(_ROWPAD - _L, _L)] = zeros
        row_dma = pltpu.make_async_copy(x_hbm, xf_v.at[pl.ds(0, _VOCAB)],
                                        row_sem)
        row_dma.start()
        pltpu.sync_copy(tgt_hbm, tgt_v)
        for j in range(_NTGT // _L):
            idx = tgt_v[pl.ds(j * _L, _L)]
            plsc.store_scatter(ids_v, [idx], j * _L + lane)
        row_dma.wait()
        pos = bcor = cnt = zeros
        for j in range(_NTGT // _L):
            occ = j * _L + lane
            idx = tgt_v[pl.ds(j * _L, _L)]
            rep = plsc.load_gather(ids_v, [idx])
            p = plsc.load_gather(xf_v, [idx])
            m = jnp.where((rep == occ) & (idx > 0), jnp.float32(1.0),
                          jnp.float32(0.0))
            pos = pos + _vlog(p + 1e-30) * m
            bcor = bcor + _neg_term(p) * m
            cnt = cnt + m
        bcor = bcor + jnp.where(lane == 0, _neg_term(xf_v[pl.ds(0, _L)]), 0.0)
        tail = zeros
        for j in range(_TAILV):
            tail = tail + _neg_term(xf_v[pl.ds(_DENSE + j * _L, _L)])
        pack_v[pl.ds(0, _L)] = pos
        pack_v[pl.ds(_L, _L)] = bcor
        pack_v[pl.ds(2 * _L, _L)] = cnt
        pack_v[pl.ds(3 * _L, _L)] = tail
        pltpu.sync_copy(pack_v, shared_v.at[pl.ds(_ND * _L, 4 * _L)])

    plsc.subcore_barrier()

    @pl.when(wid == 0)
    def _finalize():
        pltpu.sync_copy(shared_v, gath_v)
        neg = zeros
        for t in range(_ND):
            neg = neg + gath_v[pl.ds(t * _L, _L)]
        neg = neg + gath_v[pl.ds((_ND + 3) * _L, _L)]
        pos_s = jnp.full((_L,), jnp.sum(gath_v[pl.ds(_ND * _L, _L)]))
        bcor_s = jnp.full((_L,), jnp.sum(gath_v[pl.ds((_ND + 1) * _L, _L)]))
        cnt_s = jnp.full((_L,), jnp.sum(gath_v[pl.ds((_ND + 2) * _L, _L)]))
        neg_s = jnp.full((_L,), jnp.sum(neg))
        loss = (-pos_s / cnt_s
                - (neg_s - bcor_s) / (jnp.float32(_VOCAB - 1) - cnt_s))
        stage_v[...] = loss
        pltpu.sync_copy(stage_v.at[pl.ds(0, 1)], out_hbm)


_mil_kernel = functools.partial(
    pl.kernel,
    out_type=jax.ShapeDtypeStruct((1,), jnp.float32),
    mesh=plsc.VectorSubcoreMesh(core_axis_name="c", subcore_axis_name="s",
                                num_cores=1),
    compiler_params=pltpu.CompilerParams(needs_layout_passes=False),
    scratch_types=[
        pltpu.VMEM((_CHUNK,), jnp.float32),
        pltpu.VMEM((_ROWPAD,), jnp.float32),
        pltpu.VMEM((_NTGT,), jnp.int32),
        pltpu.VMEM((_VOCAB,), jnp.int32),
        pltpu.VMEM((_L,), jnp.float32),
        pltpu.VMEM((4 * _L,), jnp.float32),
        pltpu.VMEM(((_NPART + 1) * _L,), jnp.float32),
        pltpu.VMEM_SHARED(((_NPART + 1) * _L,), jnp.float32),
        pltpu.SemaphoreType.DMA,
    ],
)(_mil_body)


def kernel(input, target):
    out = _mil_kernel(input.reshape(-1), target.reshape(-1))
    return out.reshape(())


# speedup vs baseline: 1.3086x; 1.0179x over previous
import functools

import jax
import jax.numpy as jnp
from jax import lax
from jax.experimental import pallas as pl
from jax.experimental.pallas import tpu as pltpu
from jax.experimental.pallas import tpu_sc as plsc

_VOCAB = 9487
_L = 16
_ND = 14
_VPT = 42
_CHUNK = _VPT * _L
_DENSE = _ND * _CHUNK
_TAILV = 5
_ROWPAD = _DENSE + _TAILV * _L
_NTGT = 80
_NPART = _ND + 3

_LOG_COEFFS = (
    7.0376836292e-2, -1.1514610310e-1, 1.1676998740e-1, -1.2420140846e-1,
    1.4249322787e-1, -1.6668057665e-1, 2.0000714765e-1, -2.4999993993e-1,
    3.3333331174e-1,
)
_LN2 = 0.6931471805599453


def _vlog(x):
    bits = lax.bitcast_convert_type(x, jnp.int32)
    e = (bits >> 23) - 126
    m_bits = (bits & 0x007FFFFF) | 0x3F000000
    m = lax.bitcast_convert_type(m_bits, jnp.float32)
    ef = e.astype(jnp.float32)
    adj = m < jnp.float32(0.70710678)
    ef = jnp.where(adj, ef - 1.0, ef)
    t = jnp.where(adj, m + m - 1.0, m - 1.0)
    z = t * t
    p = jnp.float32(_LOG_COEFFS[0])
    for c in _LOG_COEFFS[1:]:
        p = p * t + jnp.float32(c)
    y = p * t * z - 0.5 * z
    return t + y + ef * jnp.float32(_LN2)


def _neg_term(x):
    return _vlog(1.0 - x + 1e-15)


def _mil_body(x_hbm, tgt_hbm, out_hbm, xc_v, xf_v, tgt_v, ids_v, stage_v,
              pack_v, gath_v, shared_v, row_sem):
    wid = lax.axis_index("s")
    zeros = jnp.zeros((_L,), jnp.float32)
    lane = lax.iota(jnp.int32, _L)

    @pl.when(wid < _ND)
    def _dense():
        pltpu.sync_copy(x_hbm.at[pl.ds(wid * _CHUNK, _CHUNK)], xc_v)

        def body(j, acc):
            return acc + _neg_term(xc_v[pl.ds(j * _L, _L)])

        stage_v[...] = lax.fori_loop(0, _VPT, body, zeros)
        pltpu.sync_copy(stage_v, shared_v.at[pl.ds(wid * _L, _L)])

    @pl.when(wid == _ND)
    def _sparse():
        xf_v[pl.ds(_ROWPAD - _L, _L)] = zeros
        row_dma = pltpu.make_async_copy(x_hbm, xf_v.at[pl.ds(0, _VOCAB)],
                                        row_sem)
        row_dma.start()
        pltpu.sync_copy(tgt_hbm, tgt_v)
        def scat(j, c):
            idx = tgt_v[pl.ds(j * _L, _L)]
            plsc.store_scatter(ids_v, [idx], j * _L + lane)
            return c

        lax.fori_loop(0, _NTGT // _L, scat, 0)
        row_dma.wait()

        def corr(j, carry):
            pos, bcor, cnt = carry
            occ = j * _L + lane
            idx = tgt_v[pl.ds(j * _L, _L)]
            rep = plsc.load_gather(ids_v, [idx])
            p = plsc.load_gather(xf_v, [idx])
            m = jnp.where((rep == occ) & (idx > 0), jnp.float32(1.0),
                          jnp.float32(0.0))
            return (pos + _vlog(p + 1e-30) * m, bcor + _neg_term(p) * m,
                    cnt + m)

        pos, bcor, cnt = lax.fori_loop(0, _NTGT // _L, corr,
                                       (zeros, zeros, zeros))
        bcor = bcor + jnp.where(lane == 0, _neg_term(xf_v[pl.ds(0, _L)]), 0.0)
        tail = lax.fori_loop(
            0, _TAILV,
            lambda j, acc: acc + _neg_term(xf_v[pl.ds(_DENSE + j * _L, _L)]),
            zeros)
        pack_v[pl.ds(0, _L)] = pos
        pack_v[pl.ds(_L, _L)] = bcor
        pack_v[pl.ds(2 * _L, _L)] = cnt
        pack_v[pl.ds(3 * _L, _L)] = tail
        pltpu.sync_copy(pack_v, shared_v.at[pl.ds(_ND * _L, 4 * _L)])

    plsc.subcore_barrier()

    @pl.when(wid == 0)
    def _finalize():
        pltpu.sync_copy(shared_v, gath_v)
        neg = lax.fori_loop(
            0, _ND, lambda t, acc: acc + gath_v[pl.ds(t * _L, _L)], zeros)
        neg = neg + gath_v[pl.ds((_ND + 3) * _L, _L)]
        pos_s = jnp.full((_L,), jnp.sum(gath_v[pl.ds(_ND * _L, _L)]))
        bcor_s = jnp.full((_L,), jnp.sum(gath_v[pl.ds((_ND + 1) * _L, _L)]))
        cnt_s = jnp.full((_L,), jnp.sum(gath_v[pl.ds((_ND + 2) * _L, _L)]))
        neg_s = jnp.full((_L,), jnp.sum(neg))
        loss = (-pos_s / cnt_s
                - (neg_s - bcor_s) / (jnp.float32(_VOCAB - 1) - cnt_s))
        stage_v[...] = loss
        pltpu.sync_copy(stage_v.at[pl.ds(0, 1)], out_hbm)


_mil_kernel = functools.partial(
    pl.kernel,
    out_type=jax.ShapeDtypeStruct((1,), jnp.float32),
    mesh=plsc.VectorSubcoreMesh(core_axis_name="c", subcore_axis_name="s",
                                num_cores=1),
    compiler_params=pltpu.CompilerParams(needs_layout_passes=False),
    scratch_types=[
        pltpu.VMEM((_CHUNK,), jnp.float32),
        pltpu.VMEM((_ROWPAD,), jnp.float32),
        pltpu.VMEM((_NTGT,), jnp.int32),
        pltpu.VMEM((_VOCAB,), jnp.int32),
        pltpu.VMEM((_L,), jnp.float32),
        pltpu.VMEM((4 * _L,), jnp.float32),
        pltpu.VMEM(((_NPART + 1) * _L,), jnp.float32),
        pltpu.VMEM_SHARED(((_NPART + 1) * _L,), jnp.float32),
        pltpu.SemaphoreType.DMA,
    ],
)(_mil_body)


def kernel(input, target):
    out = _mil_kernel(input.reshape(-1), target.reshape(-1))
    return out.reshape(())
